# Initial kernel scaffold; baseline (speedup 1.0000x reference)
import functools
import jax
import jax.numpy as jnp
from jax import lax
from jax.experimental import pallas as pl
from jax.experimental.pallas import tpu as pltpu
from jax.experimental.pallas import tpu_sc as plsc

NN = 10000
NR = 10240
CP = 48
EE = 320000
EP = 327680
HALF = NR // 2
ACCR = HALF + 64
CHUNK = 1024
NCH = EP // (16 * CHUNK)
DEGR = NR + 64

_MESH = plsc.VectorSubcoreMesh(
    core_axis_name="c", subcore_axis_name="s", num_cores=2, num_subcores=16)


def _zero_rows(buf, nrows):
    k = buf.shape[1] // 16
    def body(r, _):
        for j in range(k):
            buf[r, pl.ds(j * 16, 16)] = jnp.zeros((16,), jnp.float32)
        return 0
    lax.fori_loop(0, nrows, body, 0)


@functools.partial(
    pl.kernel,
    out_type=jax.ShapeDtypeStruct((2, DEGR, 16), jnp.float32),
    mesh=_MESH,
    scratch_types=[
        pltpu.VMEM((128, 16), jnp.float32),
        pltpu.VMEM((8, 128), jnp.int32),
        pltpu.VMEM_SHARED((DEGR, 16), jnp.float32),
    ],
)
def _deg_kernel(dst_hbm, degp, ebuf, dstb, acc):
    c = lax.axis_index("c")
    s = lax.axis_index("s")
    _zero_rows(ebuf, 128)
    rows_per_tile = DEGR // 16
    myz = s * rows_per_tile
    for off in (0, 128, 256, 384, 512, rows_per_tile - 128):
        pltpu.sync_copy(ebuf, acc.at[pl.ds(myz + off, 128)])
    e0 = jnp.where(lax.iota(jnp.int32, 16) == 0, 1.0, 0.0).astype(jnp.float32)
    def sete0(r, _):
        ebuf[r, pl.ds(0, 16)] = e0
        return 0
    lax.fori_loop(0, 128, sete0, 0)
    plsc.subcore_barrier()
    base_rows = (c * 16 + s) * (EP // 32 // 128)
    def chunk(g, _):
        pltpu.sync_copy(dst_hbm.at[pl.ds(base_rows + g * 8, 8)], dstb)
        for j in range(8):
            pltpu.sync_copy(ebuf, acc.at[dstb.at[j]], add=True)
        return 0
    lax.fori_loop(0, EP // 32 // CHUNK, chunk, 0)
    plsc.subcore_barrier()
    pltpu.sync_copy(acc.at[pl.ds(myz, rows_per_tile)],
                    degp.at[c, pl.ds(myz, rows_per_tile)])


def _prep_body(x_ref, w_ref, b_ref, degp_ref, i_ref,
               y0_ref, a_ref, g_ref, dx_ref):
    h = jnp.dot(x_ref[...], w_ref[...],
                preferred_element_type=jnp.float32) + b_ref[...]
    deg = degp_ref[0, :, 0] + degp_ref[1, :, 0] + 1.0
    dinv = lax.rsqrt(deg)[:, None]
    rows = i_ref[0] * 512 + lax.broadcasted_iota(jnp.int32, (512, 1), 0)
    valid = rows < NN
    zero = jnp.zeros_like(h)
    y0_ref[...] = jnp.where(valid, dinv * h, zero)
    a_ref[...] = jnp.where(valid, jnp.broadcast_to(0.9 * dinv * dinv, h.shape),
                           zero)
    g_ref[...] = jnp.where(valid, 0.1 * dinv * h, zero)
    dx_ref[...] = jnp.where(valid, jnp.broadcast_to(dinv, h.shape),
                            jnp.ones_like(h))


def _prep(x_pad, w_pad, b_pad, degp, iota):
    spec48 = pl.BlockSpec((512, CP), lambda i: (i, 0))
    return pl.pallas_call(
        _prep_body,
        grid=(NR // 512,),
        in_specs=[
            pl.BlockSpec((512, 128), lambda i: (i, 0)),
            pl.BlockSpec((128, CP), lambda i: (0, 0)),
            pl.BlockSpec((1, CP), lambda i: (0, 0)),
            pl.BlockSpec((2, 512, 16), lambda i: (0, i, 0)),
            pl.BlockSpec((1,), lambda i: (i,), memory_space=pltpu.SMEM),
        ],
        out_specs=[spec48, spec48, spec48, spec48],
        out_shape=[jax.ShapeDtypeStruct((NR, CP), jnp.float32)] * 4,
    )(x_pad, w_pad, b_pad, degp, iota)


@functools.partial(
    pl.kernel,
    out_type=jax.ShapeDtypeStruct((NR, CP), jnp.float32),
    mesh=_MESH,
    scratch_types=[
        pltpu.VMEM((CHUNK,), jnp.int32),
        pltpu.VMEM((8, 128), jnp.int32),
        pltpu.VMEM((CHUNK, CP), jnp.float32),
        pltpu.VMEM((64, CP), jnp.float32),
        pltpu.VMEM((64, CP), jnp.float32),
        pltpu.VMEM((64, CP), jnp.float32),
        pltpu.VMEM((64, CP), jnp.float32),
        pltpu.VMEM((64, CP), jnp.float32),
        pltpu.VMEM_SHARED((ACCR, CP), jnp.float32),
        pltpu.SemaphoreType.DMA,
    ],
)
def _prop_kernel(y_hbm, a_hbm, g_hbm, src_hbm, dst_hbm, ynext_hbm,
                 srcb, dstb, rows, sb, yb, ab, gb, ob, acc, sem):
    c = lax.axis_index("c")
    s = lax.axis_index("s")
    _zero_rows(ob, 64)
    zrows = ACCR // 16
    myz = s * zrows
    for off in (0, 64, 128, 192, 256, zrows - 64):
        pltpu.sync_copy(ob, acc.at[pl.ds(myz + off, 64)])
    plsc.subcore_barrier()
    ebase = s * (EP // 16)
    rbase = ebase // 128
    def chunk(gi, _):
        pltpu.sync_copy(src_hbm.at[pl.ds(ebase + gi * CHUNK, CHUNK)], srcb)
        pltpu.sync_copy(dst_hbm.at[c, pl.ds(rbase + gi * 8, 8)], dstb)
        pltpu.async_copy(y_hbm.at[srcb], rows, sem).wait()
        for j in range(8):
            pltpu.sync_copy(rows.at[pl.ds(j * 128, 128)],
                            acc.at[dstb.at[j]], add=True)
        return 0
    lax.fori_loop(0, NCH, chunk, 0)
    plsc.subcore_barrier()
    lbase = s * (HALF // 16)
    gbase = c * HALF + lbase
    for k in range(HALF // 16 // 64):
        lo = lbase + k * 64
        go = gbase + k * 64
        pltpu.sync_copy(acc.at[pl.ds(lo, 64)], sb)
        pltpu.sync_copy(y_hbm.at[pl.ds(go, 64)], yb)
        pltpu.sync_copy(a_hbm.at[pl.ds(go, 64)], ab)
        pltpu.sync_copy(g_hbm.at[pl.ds(go, 64)], gb)
        def brow(r, _):
            for j in range(3):
                cs = pl.ds(j * 16, 16)
                ob[r, cs] = ab[r, cs] * (sb[r, cs] + yb[r, cs]) + gb[r, cs]
            return 0
        lax.fori_loop(0, 64, brow, 0)
        pltpu.sync_copy(ob, ynext_hbm.at[pl.ds(go, 64)])


def _softmax_body(y_ref, dx_ref, o_ref):
    f = y_ref[...] / dx_ref[...]
    colmask = lax.broadcasted_iota(jnp.int32, (1, CP), 1) < 40
    fm = jnp.where(colmask, f, -jnp.inf)
    m = jnp.max(fm, axis=1, keepdims=True)
    e = jnp.where(colmask, jnp.exp(f - m), 0.0)
    ssum = jnp.sum(e, axis=1, keepdims=True)
    o_ref[...] = f - m - jnp.log(ssum)


def _softmax(y, dx):
    spec48 = pl.BlockSpec((512, CP), lambda i: (i, 0))
    return pl.pallas_call(
        _softmax_body,
        grid=(NR // 512,),
        in_specs=[spec48, spec48],
        out_specs=spec48,
        out_shape=jax.ShapeDtypeStruct((NR, CP), jnp.float32),
    )(y, dx)


def kernel(x, edge_index, W1, b1):
    src = edge_index[0]
    dst = edge_index[1]
    pad = EP - EE
    srcp = jnp.concatenate([src, jnp.zeros((pad,), jnp.int32)])
    dstg = jnp.concatenate([dst, jnp.full((pad,), NR, jnp.int32)])
    dst_deg = dstg.reshape(EP // 128, 128)
    halves = []
    for c in (0, 1):
        lo = c * HALF
        inr = (dstg >= lo) & (dstg < lo + HALF)
        halves.append(jnp.where(inr, dstg - lo, HALF).reshape(EP // 128, 128))
    dstp = jnp.stack(halves)

    x_pad = jnp.pad(x, ((0, NR - NN), (0, 0)))
    w_pad = jnp.pad(W1, ((0, 0), (0, CP - 40)))
    b_pad = jnp.pad(b1, (0, CP - 40)).reshape(1, CP)
    iota = jnp.arange(NR // 512, dtype=jnp.int32)

    degp = _deg_kernel(dst_deg)[:, :NR, :]
    y, a, g, dx = _prep(x_pad, w_pad, b_pad, degp, iota)
    for _ in range(10):
        y = _prop_kernel(y, a, g, srcp, dstp)
    out = _softmax(y, dx)
    return out[:NN, :40]


# speedup vs baseline: 6.4894x; 6.4894x over previous
import functools
import jax
import jax.numpy as jnp
from jax import lax
from jax.experimental import pallas as pl
from jax.experimental.pallas import tpu as pltpu
from jax.experimental.pallas import tpu_sc as plsc

NN = 10000
NR = 10240
CP = 48
EE = 320000
EP = 327680
HALF = NR // 2
ACCR = HALF + 128
CHUNK = 1024
NCH = EP // (16 * CHUNK)
DEGR = NR + 128

_MESH = plsc.VectorSubcoreMesh(
    core_axis_name="c", subcore_axis_name="s", num_cores=2, num_subcores=16)
_SC_PARAMS = pltpu.CompilerParams(use_tc_tiling_on_sc=False)


def _zero_rows(buf, nrows):
    k = buf.shape[1] // 16
    def body(r, _):
        for j in range(k):
            buf[r, pl.ds(j * 16, 16)] = jnp.zeros((16,), jnp.float32)
        return 0
    lax.fori_loop(0, nrows, body, 0)


@functools.partial(
    pl.kernel,
    out_type=jax.ShapeDtypeStruct((2, DEGR, 16), jnp.float32),
    mesh=_MESH,
    compiler_params=_SC_PARAMS,
    scratch_types=[
        pltpu.VMEM((128, 16), jnp.float32),
        pltpu.VMEM((8, 128), jnp.int32),
        pltpu.VMEM_SHARED((DEGR, 16), jnp.float32),
    ],
)
def _deg_kernel(dst_hbm, degp, ebuf, dstb, acc):
    c = lax.axis_index("c")
    s = lax.axis_index("s")
    _zero_rows(ebuf, 128)
    rows_per_tile = DEGR // 16
    myz = s * rows_per_tile
    for off in (0, 128, 256, 384, 512, rows_per_tile - 128):
        pltpu.sync_copy(ebuf, acc.at[pl.ds(myz + off, 128)])
    e0 = jnp.where(lax.iota(jnp.int32, 16) == 0, 1.0, 0.0).astype(jnp.float32)
    def sete0(r, _):
        ebuf[r, pl.ds(0, 16)] = e0
        return 0
    lax.fori_loop(0, 128, sete0, 0)
    plsc.subcore_barrier()
    base_rows = (c * 16 + s) * (EP // 32 // 128)
    def chunk(g, _):
        pltpu.sync_copy(
            dst_hbm.at[pl.ds(pl.multiple_of(base_rows + g * 8, 8), 8)], dstb)
        for j in range(8):
            pltpu.sync_copy(ebuf, acc.at[dstb.at[j]], add=True)
        return 0
    lax.fori_loop(0, EP // 32 // CHUNK, chunk, 0)
    plsc.subcore_barrier()
    myz8 = pl.multiple_of(myz, 8)
    pltpu.sync_copy(acc.at[pl.ds(myz8, rows_per_tile)],
                    degp.at[c, pl.ds(myz8, rows_per_tile)])


def _prep_body(x_ref, w_ref, b_ref, degp_ref,
               y0_ref, a_ref, g_ref, dx_ref):
    h = jnp.dot(x_ref[...], w_ref[...],
                preferred_element_type=jnp.float32) + b_ref[...]
    deg = degp_ref[0, :, 0] + degp_ref[1, :, 0] + 1.0
    dinv = lax.rsqrt(deg)[:, None]
    rows = pl.program_id(0) * 512 + lax.broadcasted_iota(jnp.int32, (512, 1), 0)
    valid = rows < NN
    zero = jnp.zeros_like(h)
    y0_ref[...] = jnp.where(valid, dinv * h, zero)
    a_ref[...] = jnp.where(valid, jnp.broadcast_to(0.9 * dinv * dinv, h.shape),
                           zero)
    g_ref[...] = jnp.where(valid, 0.1 * dinv * h, zero)
    dx_ref[...] = jnp.where(valid, jnp.broadcast_to(dinv, h.shape),
                            jnp.ones_like(h))


def _prep(x_pad, w_pad, b_pad, degp):
    spec48 = pl.BlockSpec((512, CP), lambda i: (i, 0))
    return pl.pallas_call(
        _prep_body,
        grid=(NR // 512,),
        in_specs=[
            pl.BlockSpec((512, 128), lambda i: (i, 0)),
            pl.BlockSpec((128, CP), lambda i: (0, 0)),
            pl.BlockSpec((1, CP), lambda i: (0, 0)),
            pl.BlockSpec((2, 512, 16), lambda i: (0, i, 0)),
        ],
        out_specs=[spec48, spec48, spec48, spec48],
        out_shape=[jax.ShapeDtypeStruct((NR, CP), jnp.float32)] * 4,
    )(x_pad, w_pad, b_pad, degp)


@functools.partial(
    pl.kernel,
    out_type=jax.ShapeDtypeStruct((NR, CP), jnp.float32),
    mesh=_MESH,
    compiler_params=_SC_PARAMS,
    scratch_types=[
        pltpu.VMEM((CHUNK,), jnp.int32),
        pltpu.VMEM((8, 128), jnp.int32),
        pltpu.VMEM((CHUNK, CP), jnp.float32),
        pltpu.VMEM((64, CP), jnp.float32),
        pltpu.VMEM((64, CP), jnp.float32),
        pltpu.VMEM((64, CP), jnp.float32),
        pltpu.VMEM((64, CP), jnp.float32),
        pltpu.VMEM((64, CP), jnp.float32),
        pltpu.VMEM_SHARED((ACCR, CP), jnp.float32),
        pltpu.SemaphoreType.DMA,
    ],
)
def _prop_kernel(y_hbm, a_hbm, g_hbm, src_hbm, dst_hbm, ynext_hbm,
                 srcb, dstb, rows, sb, yb, ab, gb, ob, acc, sem):
    c = lax.axis_index("c")
    s = lax.axis_index("s")
    _zero_rows(ob, 64)
    zrows = ACCR // 16
    myz = s * zrows
    for off in (0, 64, 128, 192, 256, zrows - 64):
        pltpu.sync_copy(ob, acc.at[pl.ds(pl.multiple_of(myz + off, 8), 64)])
    plsc.subcore_barrier()
    ebase = s * (EP // 16)
    rbase = ebase // 128
    def chunk(gi, _):
        pltpu.sync_copy(
            src_hbm.at[pl.ds(pl.multiple_of(ebase + gi * CHUNK, 1024), CHUNK)],
            srcb)
        pltpu.sync_copy(
            dst_hbm.at[c, pl.ds(pl.multiple_of(rbase + gi * 8, 8), 8)], dstb)
        pltpu.async_copy(y_hbm.at[srcb], rows, sem).wait()
        for j in range(8):
            pltpu.sync_copy(rows.at[pl.ds(j * 128, 128)],
                            acc.at[dstb.at[j]], add=True)
        return 0
    lax.fori_loop(0, NCH, chunk, 0)
    plsc.subcore_barrier()
    lbase = s * (HALF // 16)
    gbase = c * HALF + lbase
    for k in range(HALF // 16 // 64):
        lo = pl.multiple_of(lbase + k * 64, 64)
        go = pl.multiple_of(gbase + k * 64, 64)
        pltpu.sync_copy(acc.at[pl.ds(lo, 64)], sb)
        pltpu.sync_copy(y_hbm.at[pl.ds(go, 64)], yb)
        pltpu.sync_copy(a_hbm.at[pl.ds(go, 64)], ab)
        pltpu.sync_copy(g_hbm.at[pl.ds(go, 64)], gb)
        def brow(r, _):
            for j in range(3):
                cs = pl.ds(j * 16, 16)
                ob[r, cs] = ab[r, cs] * (sb[r, cs] + yb[r, cs]) + gb[r, cs]
            return 0
        lax.fori_loop(0, 64, brow, 0)
        pltpu.sync_copy(ob, ynext_hbm.at[pl.ds(go, 64)])


def _softmax_body(y_ref, dx_ref, o_ref):
    f = y_ref[...] / dx_ref[...]
    colmask = lax.broadcasted_iota(jnp.int32, (1, CP), 1) < 40
    fm = jnp.where(colmask, f, -jnp.inf)
    m = jnp.max(fm, axis=1, keepdims=True)
    e = jnp.where(colmask, jnp.exp(f - m), 0.0)
    ssum = jnp.sum(e, axis=1, keepdims=True)
    o_ref[...] = f - m - jnp.log(ssum)


def _softmax(y, dx):
    spec48 = pl.BlockSpec((512, CP), lambda i: (i, 0))
    return pl.pallas_call(
        _softmax_body,
        grid=(NR // 512,),
        in_specs=[spec48, spec48],
        out_specs=spec48,
        out_shape=jax.ShapeDtypeStruct((NR, CP), jnp.float32),
    )(y, dx)


def kernel(x, edge_index, W1, b1):
    src = edge_index[0]
    dst = edge_index[1]
    pad = EP - EE
    srcp = jnp.concatenate([src, jnp.zeros((pad,), jnp.int32)])
    dstg = jnp.concatenate([dst, jnp.full((pad,), NR, jnp.int32)])
    dst_deg = dstg.reshape(EP // 128, 128)
    halves = []
    for c in (0, 1):
        lo = c * HALF
        inr = (dstg >= lo) & (dstg < lo + HALF)
        halves.append(jnp.where(inr, dstg - lo, HALF).reshape(EP // 128, 128))
    dstp = jnp.stack(halves)

    x_pad = jnp.pad(x, ((0, NR - NN), (0, 0)))
    w_pad = jnp.pad(W1, ((0, 0), (0, CP - 40)))
    b_pad = jnp.pad(b1, (0, CP - 40)).reshape(1, CP)
    degp = _deg_kernel(dst_deg)[:, :NR, :]
    y, a, g, dx = _prep(x_pad, w_pad, b_pad, degp)
    for _ in range(10):
        y = _prop_kernel(y, a, g, srcp, dstp)
    out = _softmax(y, dx)
    return out[:NN, :40]


# speedup vs baseline: 6.8299x; 1.0525x over previous
import functools
import jax
import jax.numpy as jnp
from jax import lax
from jax.experimental import pallas as pl
from jax.experimental.pallas import tpu as pltpu
from jax.experimental.pallas import tpu_sc as plsc

NN = 10000
NR = 10240
CP = 48
EE = 320000
EP = 327680
HALF = NR // 2
ACCR = HALF + 128
CHUNK = 1024
NCH = EP // (16 * CHUNK)
DEGR = NR + 128

_MESH = plsc.VectorSubcoreMesh(
    core_axis_name="c", subcore_axis_name="s", num_cores=2, num_subcores=16)
_SC_PARAMS = pltpu.CompilerParams(use_tc_tiling_on_sc=False)


def _zero_rows(buf, nrows):
    k = buf.shape[1] // 16
    def body(r, _):
        for j in range(k):
            buf[r, pl.ds(j * 16, 16)] = jnp.zeros((16,), jnp.float32)
        return 0
    lax.fori_loop(0, nrows, body, 0)


@functools.partial(
    pl.kernel,
    out_type=jax.ShapeDtypeStruct((2, DEGR, 16), jnp.float32),
    mesh=_MESH,
    compiler_params=_SC_PARAMS,
    scratch_types=[
        pltpu.VMEM((128, 16), jnp.float32),
        pltpu.VMEM((8, 128), jnp.int32),
        pltpu.VMEM_SHARED((DEGR, 16), jnp.float32),
    ],
)
def _deg_kernel(dst_hbm, degp, ebuf, dstb, acc):
    c = lax.axis_index("c")
    s = lax.axis_index("s")
    _zero_rows(ebuf, 128)
    rows_per_tile = DEGR // 16
    myz = s * rows_per_tile
    for off in (0, 128, 256, 384, 512, rows_per_tile - 128):
        pltpu.sync_copy(ebuf, acc.at[pl.ds(myz + off, 128)])
    e0 = jnp.where(lax.iota(jnp.int32, 16) == 0, 1.0, 0.0).astype(jnp.float32)
    def sete0(r, _):
        ebuf[r, pl.ds(0, 16)] = e0
        return 0
    lax.fori_loop(0, 128, sete0, 0)
    plsc.subcore_barrier()
    base_rows = (c * 16 + s) * (EP // 32 // 128)
    def chunk(g, _):
        pltpu.sync_copy(
            dst_hbm.at[pl.ds(pl.multiple_of(base_rows + g * 8, 8), 8)], dstb)
        for j in range(8):
            pltpu.sync_copy(ebuf, acc.at[dstb.at[j]], add=True)
        return 0
    lax.fori_loop(0, EP // 32 // CHUNK, chunk, 0)
    plsc.subcore_barrier()
    myz8 = pl.multiple_of(myz, 8)
    pltpu.sync_copy(acc.at[pl.ds(myz8, rows_per_tile)],
                    degp.at[c, pl.ds(myz8, rows_per_tile)])


def _prep_body(x_ref, w_ref, b_ref, degp_ref,
               y0_ref, a_ref, g_ref, dx_ref):
    h = jnp.dot(x_ref[...], w_ref[...],
                preferred_element_type=jnp.float32) + b_ref[...]
    deg = degp_ref[0, :, 0] + degp_ref[1, :, 0] + 1.0
    dinv = lax.rsqrt(deg)[:, None]
    rows = pl.program_id(0) * 512 + lax.broadcasted_iota(jnp.int32, (512, 1), 0)
    valid = rows < NN
    zero = jnp.zeros_like(h)
    y0_ref[...] = jnp.where(valid, dinv * h, zero)
    a_ref[...] = jnp.where(valid, jnp.broadcast_to(0.9 * dinv * dinv, h.shape),
                           zero)
    g_ref[...] = jnp.where(valid, 0.1 * dinv * h, zero)
    dx_ref[...] = jnp.where(valid, jnp.broadcast_to(dinv, h.shape),
                            jnp.ones_like(h))


def _prep(x_pad, w_pad, b_pad, degp):
    spec48 = pl.BlockSpec((512, CP), lambda i: (i, 0))
    return pl.pallas_call(
        _prep_body,
        grid=(NR // 512,),
        in_specs=[
            pl.BlockSpec((512, 128), lambda i: (i, 0)),
            pl.BlockSpec((128, CP), lambda i: (0, 0)),
            pl.BlockSpec((1, CP), lambda i: (0, 0)),
            pl.BlockSpec((2, 512, 16), lambda i: (0, i, 0)),
        ],
        out_specs=[spec48, spec48, spec48, spec48],
        out_shape=[jax.ShapeDtypeStruct((NR, CP), jnp.float32)] * 4,
    )(x_pad, w_pad, b_pad, degp)


@functools.partial(
    pl.kernel,
    out_type=jax.ShapeDtypeStruct((NR, CP), jnp.float32),
    mesh=_MESH,
    compiler_params=_SC_PARAMS,
    scratch_types=[
        pltpu.VMEM((2, CHUNK), jnp.int32),
        pltpu.VMEM((2, 8, 128), jnp.int32),
        pltpu.VMEM((2, CHUNK, CP), jnp.float32),
        pltpu.VMEM_SHARED((ACCR, CP), jnp.float32),
        pltpu.SemaphoreType.DMA,
        pltpu.SemaphoreType.DMA,
        pltpu.SemaphoreType.DMA,
    ],
)
def _prop_kernel(y_hbm, a_hbm, g_hbm, src_hbm, dst_hbm, ynext_hbm,
                 srcb, dstb, rows, acc, sem_g0, sem_g1, sem_s):
    c = lax.axis_index("c")
    s = lax.axis_index("s")
    sems = (sem_g0, sem_g1)
    sb = rows.at[0].at[pl.ds(0, 64)]
    yb = rows.at[0].at[pl.ds(64, 64)]
    ab = rows.at[0].at[pl.ds(128, 64)]
    gb = rows.at[0].at[pl.ds(192, 64)]
    ob = rows.at[0].at[pl.ds(256, 64)]
    _zero_rows(ob, 64)
    zrows = ACCR // 16
    myz = s * zrows
    for off in (0, 64, 128, 192, 256, zrows - 64):
        pltpu.sync_copy(ob, acc.at[pl.ds(pl.multiple_of(myz + off, 8), 64)])
    plsc.subcore_barrier()
    ebase = s * (EP // 16)
    rbase = ebase // 128

    def load_and_fire(b, g):
        pltpu.sync_copy(
            src_hbm.at[pl.ds(pl.multiple_of(ebase + g * CHUNK, 1024), CHUNK)],
            srcb.at[b])
        pltpu.sync_copy(
            dst_hbm.at[c, pl.ds(pl.multiple_of(rbase + g * 8, 8), 8)],
            dstb.at[b])
        pltpu.async_copy(y_hbm.at[srcb.at[b]], rows.at[b], sems[b])

    for b in range(2):
        load_and_fire(b, b)

    def pair(gi, _):
        for b in range(2):
            g = 2 * gi + b
            pltpu.make_async_copy(y_hbm.at[srcb.at[b]], rows.at[b],
                                  sems[b]).wait()
            descs = [
                pltpu.async_copy(rows.at[b].at[pl.ds(j * 128, 128)],
                                 acc.at[dstb.at[b].at[j]], sem_s, add=True)
                for j in range(8)
            ]
            for d in descs:
                d.wait()

            @pl.when(g + 2 < NCH)
            def _():
                load_and_fire(b, g + 2)
        return 0
    lax.fori_loop(0, NCH // 2, pair, 0)
    plsc.subcore_barrier()
    lbase = s * (HALF // 16)
    gbase = c * HALF + lbase
    for k in range(HALF // 16 // 64):
        lo = pl.multiple_of(lbase + k * 64, 64)
        go = pl.multiple_of(gbase + k * 64, 64)
        pltpu.sync_copy(acc.at[pl.ds(lo, 64)], sb)
        pltpu.sync_copy(y_hbm.at[pl.ds(go, 64)], yb)
        pltpu.sync_copy(a_hbm.at[pl.ds(go, 64)], ab)
        pltpu.sync_copy(g_hbm.at[pl.ds(go, 64)], gb)
        def brow(r, _):
            for j in range(3):
                cs = pl.ds(j * 16, 16)
                ob[r, cs] = ab[r, cs] * (sb[r, cs] + yb[r, cs]) + gb[r, cs]
            return 0
        lax.fori_loop(0, 64, brow, 0)
        pltpu.sync_copy(ob, ynext_hbm.at[pl.ds(go, 64)])


def _softmax_body(y_ref, dx_ref, o_ref):
    f = y_ref[...] / dx_ref[...]
    colmask = lax.broadcasted_iota(jnp.int32, (1, CP), 1) < 40
    fm = jnp.where(colmask, f, -jnp.inf)
    m = jnp.max(fm, axis=1, keepdims=True)
    e = jnp.where(colmask, jnp.exp(f - m), 0.0)
    ssum = jnp.sum(e, axis=1, keepdims=True)
    o_ref[...] = f - m - jnp.log(ssum)


def _softmax(y, dx):
    spec48 = pl.BlockSpec((512, CP), lambda i: (i, 0))
    return pl.pallas_call(
        _softmax_body,
        grid=(NR // 512,),
        in_specs=[spec48, spec48],
        out_specs=spec48,
        out_shape=jax.ShapeDtypeStruct((NR, CP), jnp.float32),
    )(y, dx)


def kernel(x, edge_index, W1, b1):
    src = edge_index[0]
    dst = edge_index[1]
    pad = EP - EE
    srcp = jnp.concatenate([src, jnp.zeros((pad,), jnp.int32)])
    dstg = jnp.concatenate([dst, jnp.full((pad,), NR, jnp.int32)])
    dst_deg = dstg.reshape(EP // 128, 128)
    halves = []
    for c in (0, 1):
        lo = c * HALF
        inr = (dstg >= lo) & (dstg < lo + HALF)
        halves.append(jnp.where(inr, dstg - lo, HALF).reshape(EP // 128, 128))
    dstp = jnp.stack(halves)

    x_pad = jnp.pad(x, ((0, NR - NN), (0, 0)))
    w_pad = jnp.pad(W1, ((0, 0), (0, CP - 40)))
    b_pad = jnp.pad(b1, (0, CP - 40)).reshape(1, CP)
    degp = _deg_kernel(dst_deg)[:, :NR, :]
    y, a, g, dx = _prep(x_pad, w_pad, b_pad, degp)
    for _ in range(10):
        y = _prop_kernel(y, a, g, srcp, dstp)
    out = _softmax(y, dx)
    return out[:NN, :40]
